# baseline (speedup 1.0000x reference)
import jax
import jax.numpy as jnp
from jax import lax
from jax.experimental import pallas as pl
from jax.experimental.pallas import tpu as pltpu
from jax.experimental.pallas import tpu_sc as plsc

_B = 16384
_D = 16
_NC = 2
_NS = 16
_NW = _NC * _NS
_BPW = _B // _NW
_CH = 128
_NCH = _BPW // _CH
_R = 2 * _D * _NCH


def _gather_body(tu, ti, uids, iids, out, idx, rows, sem):
    wid = lax.axis_index("s") * _NC + lax.axis_index("c")
    pltpu.sync_copy(uids.at[wid], idx.at[pl.ds(0, _NCH)])
    pltpu.sync_copy(iids.at[wid], idx.at[pl.ds(_NCH, _NCH)])

    def step(d, carry):
        copies = []
        for t, tbl in ((0, tu), (1, ti)):
            for j in range(_NCH):
                copies.append(
                    pltpu.async_copy(
                        tbl.at[d].at[idx.at[t * _NCH + j]],
                        rows.at[(t * _D + d) * _NCH + j],
                        sem,
                    ))
        for c in copies:
            c.wait()
        return carry

    lax.fori_loop(0, _D, step, 0)
    pltpu.sync_copy(rows, out.at[wid])


def _sc_gather(tu, ti, uids, iids):
    mesh = plsc.VectorSubcoreMesh(core_axis_name="c", subcore_axis_name="s")
    f = pl.kernel(
        _gather_body,
        out_type=jax.ShapeDtypeStruct((_NW, _R, _CH), jnp.float32),
        mesh=mesh,
        scratch_types=[
            pltpu.VMEM((2 * _NCH, _CH), jnp.int32),
            pltpu.VMEM((_R, _CH), jnp.float32),
            pltpu.SemaphoreType.DMA,
        ],
        compiler_params=pltpu.CompilerParams(use_tc_tiling_on_sc=False),
    )
    return f(tu, ti, uids, iids)


def _mlp_body(x_ref, w1_ref, b1_ref, gamma_ref, beta_ref, w2_ref, b2_ref,
              out_ref):
    x = x_ref[...]
    h = lax.dot_general(w1_ref[...], x, (((0,), (0,)), ((), ())),
                        preferred_element_type=jnp.float32)
    h = h + b1_ref[...]
    h = jnp.maximum(h, 0.0)
    mean = jnp.mean(h, axis=1, keepdims=True)
    c = h - mean
    var = jnp.mean(c * c, axis=1, keepdims=True)
    hn = c * lax.rsqrt(var + 1e-5) * gamma_ref[...] + beta_ref[...]
    logit = lax.dot_general(w2_ref[...], hn, (((0,), (0,)), ((), ())),
                            preferred_element_type=jnp.float32)
    out_ref[...] = 1.0 / (1.0 + jnp.exp(-(logit + b2_ref[0])))


def _tc_mlp(x, W1, b1, gamma, beta, W2, b2):
    return pl.pallas_call(
        _mlp_body,
        out_shape=jax.ShapeDtypeStruct((1, _B), jnp.float32),
        in_specs=[
            pl.BlockSpec(memory_space=pltpu.VMEM),
            pl.BlockSpec(memory_space=pltpu.VMEM),
            pl.BlockSpec(memory_space=pltpu.VMEM),
            pl.BlockSpec(memory_space=pltpu.VMEM),
            pl.BlockSpec(memory_space=pltpu.VMEM),
            pl.BlockSpec(memory_space=pltpu.VMEM),
            pl.BlockSpec(memory_space=pltpu.SMEM),
        ],
        out_specs=pl.BlockSpec(memory_space=pltpu.VMEM),
    )(x, W1, b1.reshape(_D, 1), gamma.reshape(_D, 1), beta.reshape(_D, 1),
      W2, b2)


def kernel(user_id, item_id, user_table, item_table, W1, b1, gamma, beta,
           W2, b2):
    uids = user_id.reshape(_NW, _NCH, _CH)
    iids = item_id.reshape(_NW, _NCH, _CH)
    g = _sc_gather(user_table.T, item_table.T, uids, iids)
    x = (g.reshape(_NW, 2 * _D, _NCH, _CH)
          .transpose(1, 0, 2, 3)
          .reshape(2 * _D, _B))
    y = _tc_mlp(x, W1, b1, gamma, beta, W2, b2)
    return y.reshape(_B, 1)
